# baseline (speedup 1.0000x reference)
import jax
import jax.numpy as jnp
from jax import lax
from jax.experimental import pallas as pl
from jax.experimental.pallas import tpu as pltpu
from jax.experimental.pallas import tpu_sc as plsc

NC = 2
NS = 16
NW = NC * NS

ACTION_SIZE = 1_000_000
FEATURE_DIM = 64
BATCH = 16384
HIST = 200

TOT = BATCH * HIST
IDX_MINOR = 128
ROWS = TOT // IDX_MINOR
ROWS_PER_W = ROWS // NW
CHUNK_ROWS = 4
CHUNK = CHUNK_ROWS * IDX_MINOR
OUT_ROWS_PER_CHUNK = CHUNK // 2
N_CHUNKS = ROWS_PER_W // CHUNK_ROWS
N_ITERS = N_CHUNKS // 2


def _body(idx_hbm, table_hbm, out_hbm,
          idx0, idx1, rows0, rows1,
          isem0, isem1, gsem0, gsem1, ssem0, ssem1):
    wid = lax.axis_index("s") * NC + lax.axis_index("c")
    r0 = wid * ROWS_PER_W

    def idx_copy(c, buf, sem):
        pltpu.async_copy(idx_hbm.at[pl.ds(r0 + c * CHUNK_ROWS, CHUNK_ROWS)],
                         buf, sem)

    def wait_idx(buf, sem):
        pltpu.make_async_copy(idx_hbm.at[pl.ds(r0, CHUNK_ROWS)], buf, sem).wait()

    def gathers(idxbuf, rowbuf, sem):
        for j in range(CHUNK_ROWS):
            pltpu.async_copy(
                table_hbm.at[idxbuf.at[j]],
                rowbuf.at[pl.ds(j * IDX_MINOR, IDX_MINOR)],
                sem,
            )

    def wait_gathers(rowbuf, sem):
        pltpu.make_async_copy(table_hbm.at[pl.ds(0, CHUNK)], rowbuf, sem).wait()

    def scatter(c, rowbuf, sem):
        base = (r0 + c * CHUNK_ROWS) * IDX_MINOR // 2
        pltpu.async_copy(
            rowbuf.at[pl.ds(0, OUT_ROWS_PER_CHUNK)],
            out_hbm.at[pl.ds(base, OUT_ROWS_PER_CHUNK), pl.ds(0, FEATURE_DIM)],
            sem,
        )
        pltpu.async_copy(
            rowbuf.at[pl.ds(OUT_ROWS_PER_CHUNK, OUT_ROWS_PER_CHUNK)],
            out_hbm.at[pl.ds(base, OUT_ROWS_PER_CHUNK),
                       pl.ds(FEATURE_DIM, FEATURE_DIM)],
            sem,
        )

    def wait_scatter(rowbuf, sem):
        pltpu.make_async_copy(table_hbm.at[pl.ds(0, CHUNK)], rowbuf, sem).wait()

    idx_copy(0, idx0, isem0)
    wait_idx(idx0, isem0)
    gathers(idx0, rows0, gsem0)
    idx_copy(1, idx1, isem1)

    def step(i, _):
        c0 = 2 * i
        c1 = c0 + 1
        wait_idx(idx1, isem1)

        @pl.when(i > 0)
        def _():
            wait_scatter(rows1, ssem1)

        gathers(idx1, rows1, gsem1)
        wait_gathers(rows0, gsem0)
        scatter(c0, rows0, ssem0)

        @pl.when(i < N_ITERS - 1)
        def _():
            idx_copy(c0 + 2, idx0, isem0)

        wait_scatter(rows0, ssem0)

        @pl.when(i < N_ITERS - 1)
        def _():
            wait_idx(idx0, isem0)
            gathers(idx0, rows0, gsem0)

        wait_gathers(rows1, gsem1)
        scatter(c1, rows1, ssem1)

        @pl.when(i < N_ITERS - 1)
        def _():
            idx_copy(c1 + 2, idx1, isem1)

        return ()

    lax.fori_loop(0, N_ITERS, step, ())

    wait_scatter(rows1, ssem1)


@jax.jit
def kernel(x, wte):
    xf = (x.reshape(-1, CHUNK // 2, 2)
           .transpose(0, 2, 1)
           .reshape(ROWS, IDX_MINOR)
           .astype(jnp.int32))
    mesh = plsc.VectorSubcoreMesh(core_axis_name="c", subcore_axis_name="s")
    out = pl.kernel(
        _body,
        out_type=jax.ShapeDtypeStruct((TOT // 2, 2 * FEATURE_DIM), jnp.float32),
        mesh=mesh,
        compiler_params=pltpu.CompilerParams(use_tc_tiling_on_sc=False),
        scratch_types=[
            pltpu.VMEM((CHUNK_ROWS, IDX_MINOR), jnp.int32),
            pltpu.VMEM((CHUNK_ROWS, IDX_MINOR), jnp.int32),
            pltpu.VMEM((CHUNK, FEATURE_DIM), jnp.float32),
            pltpu.VMEM((CHUNK, FEATURE_DIM), jnp.float32),
            pltpu.SemaphoreType.DMA,
            pltpu.SemaphoreType.DMA,
            pltpu.SemaphoreType.DMA,
            pltpu.SemaphoreType.DMA,
            pltpu.SemaphoreType.DMA,
            pltpu.SemaphoreType.DMA,
        ],
    )(xf, wte)
    return out.reshape(BATCH, HIST, FEATURE_DIM)


# speedup vs baseline: 1.9831x; 1.9831x over previous
import jax
import jax.numpy as jnp
from jax import lax
from jax.experimental import pallas as pl
from jax.experimental.pallas import tpu as pltpu
from jax.experimental.pallas import tpu_sc as plsc

NC = 2
NS = 16
NW = NC * NS

ACTION_SIZE = 1_000_000
FEATURE_DIM = 64
BATCH = 16384
HIST = 200

TOT = BATCH * HIST
IDX_MINOR = 128
ROWS = TOT // IDX_MINOR
ROWS_PER_W = ROWS // NW
CHUNK_ROWS = 4
CHUNK = CHUNK_ROWS * IDX_MINOR
OUT_ROWS_PER_CHUNK = CHUNK // 2
N_CHUNKS = ROWS_PER_W // CHUNK_ROWS
N_ITERS = N_CHUNKS // 2


def _body(idx_hbm, table_hbm, out_hbm,
          idx0, idx1, rows0, rows1,
          isem0, isem1, gsem0, gsem1, ssem0, ssem1):
    wid = lax.axis_index("s") * NC + lax.axis_index("c")
    r0 = wid * ROWS_PER_W

    def idx_copy(c, buf, sem):
        pltpu.async_copy(idx_hbm.at[pl.ds(r0 + c * CHUNK_ROWS, CHUNK_ROWS)],
                         buf, sem)

    def wait_idx(buf, sem):
        pltpu.make_async_copy(idx_hbm.at[pl.ds(r0, CHUNK_ROWS)], buf, sem).wait()

    def gathers(idxbuf, rowbuf, sem):
        for j in range(CHUNK_ROWS):
            pltpu.async_copy(
                table_hbm.at[idxbuf.at[j]],
                rowbuf.at[pl.ds(j * IDX_MINOR, IDX_MINOR)],
                sem,
            )

    def wait_gathers(rowbuf, sem):
        pltpu.make_async_copy(table_hbm.at[pl.ds(0, CHUNK)], rowbuf, sem).wait()

    def scatter(c, rowbuf, sem):
        base = (r0 + c * CHUNK_ROWS) * IDX_MINOR
        pltpu.async_copy(
            rowbuf,
            out_hbm.at[pl.ds(base, CHUNK), pl.ds(0, FEATURE_DIM)],
            sem,
        )

    def wait_scatter(rowbuf, sem):
        pltpu.make_async_copy(table_hbm.at[pl.ds(0, CHUNK)], rowbuf, sem).wait()

    idx_copy(0, idx0, isem0)
    wait_idx(idx0, isem0)
    gathers(idx0, rows0, gsem0)
    idx_copy(1, idx1, isem1)

    def step(i, _):
        c0 = 2 * i
        c1 = c0 + 1
        wait_idx(idx1, isem1)

        @pl.when(i > 0)
        def _():
            wait_scatter(rows1, ssem1)

        gathers(idx1, rows1, gsem1)
        wait_gathers(rows0, gsem0)
        scatter(c0, rows0, ssem0)

        @pl.when(i < N_ITERS - 1)
        def _():
            idx_copy(c0 + 2, idx0, isem0)

        wait_scatter(rows0, ssem0)

        @pl.when(i < N_ITERS - 1)
        def _():
            wait_idx(idx0, isem0)
            gathers(idx0, rows0, gsem0)

        wait_gathers(rows1, gsem1)
        scatter(c1, rows1, ssem1)

        @pl.when(i < N_ITERS - 1)
        def _():
            idx_copy(c1 + 2, idx1, isem1)

        return ()

    lax.fori_loop(0, N_ITERS, step, ())

    wait_scatter(rows1, ssem1)


@jax.jit
def kernel(x, wte):
    xf = x.reshape(ROWS, IDX_MINOR).astype(jnp.int32)
    mesh = plsc.VectorSubcoreMesh(core_axis_name="c", subcore_axis_name="s")
    out = pl.kernel(
        _body,
        out_type=jax.ShapeDtypeStruct((TOT, 2 * FEATURE_DIM), jnp.float32),
        mesh=mesh,
        compiler_params=pltpu.CompilerParams(use_tc_tiling_on_sc=False),
        scratch_types=[
            pltpu.VMEM((CHUNK_ROWS, IDX_MINOR), jnp.int32),
            pltpu.VMEM((CHUNK_ROWS, IDX_MINOR), jnp.int32),
            pltpu.VMEM((CHUNK, FEATURE_DIM), jnp.float32),
            pltpu.VMEM((CHUNK, FEATURE_DIM), jnp.float32),
            pltpu.SemaphoreType.DMA,
            pltpu.SemaphoreType.DMA,
            pltpu.SemaphoreType.DMA,
            pltpu.SemaphoreType.DMA,
            pltpu.SemaphoreType.DMA,
            pltpu.SemaphoreType.DMA,
        ],
    )(xf, wte)
    return out[:, :FEATURE_DIM].reshape(BATCH, HIST, FEATURE_DIM)
